# baseline (speedup 1.0000x reference)
import functools
import math

import jax
import jax.numpy as jnp
from jax import lax
from jax.experimental import pallas as pl
from jax.experimental.pallas import tpu as pltpu
from jax.experimental.pallas import tpu_sc as plsc

SPECIAL_EVENT = 1
V, D, NCAT, CATK, NNUM, NTIME, B, L = 1000000, 64, 8, 1000, 16, 8, 1024, 200
BL = B * L
SCALE = math.sqrt(D)

NC, NS = 2, 16
NW = NC * NS
TOK_PER_W = BL // NW
CH = 400
NCHUNK = TOK_PER_W // CH
NG = CH // 16
SUB = 80
NSUB = CH // SUB
BPW = B // NW


_BB = 64


def _mlp_body(nf, tf, w1, b1, w2, b2, wt1, bt1, wt2, bt2, ttab, out):
    dot = lambda x, w: lax.dot_general(
        x, w[...], (((2,), (0,)), ((), ())),
        preferred_element_type=jnp.float32)
    h = jnp.maximum(dot(nf[...], w1) + b1[...], 0.0)
    num = dot(h, w2) + b2[...]
    ht = jnp.maximum(dot(tf[...], wt1) + bt1[...], 0.0)
    tim = dot(ht, wt2) + bt2[...]
    out[...] = (num + tim + ttab[1:2, :]) * SCALE


def _mlp_t8(nf, tf, W1, b1, W2, b2, Wt1, bt1, Wt2, bt2, ttab):
    grid = (B // _BB,)
    full = lambda i: (0, 0)
    return pl.pallas_call(
        _mlp_body,
        grid=grid,
        in_specs=[
            pl.BlockSpec((_BB, L, NNUM), lambda i: (i, 0, 0)),
            pl.BlockSpec((_BB, L, NTIME), lambda i: (i, 0, 0)),
            pl.BlockSpec((NNUM, D), full),
            pl.BlockSpec((1, D), full),
            pl.BlockSpec((D, D), full),
            pl.BlockSpec((1, D), full),
            pl.BlockSpec((NTIME, D), full),
            pl.BlockSpec((1, D), full),
            pl.BlockSpec((D, D), full),
            pl.BlockSpec((1, D), full),
            pl.BlockSpec((2, D), full),
        ],
        out_specs=pl.BlockSpec((_BB, L, D), lambda i: (i, 0, 0)),
        out_shape=jax.ShapeDtypeStruct((B, L, D), jnp.float32),
    )(nf, tf, W1, b1, W2, b2, Wt1, bt1, Wt2, bt2, ttab)


def _sc_body(tok_idx_hbm, catf_hbm, t8_hbm, ttable_hbm, cflat_hbm, ttab_hbm,
             out_hbm, idx_all, base_a, base_b, out_a, out_b, evcatf,
             evcat_idx, evcats, evt8, trow, gsem_a, gsem_b, esem):
    wid = lax.axis_index("s") * NC + lax.axis_index("c")
    wbase = wid * TOK_PER_W

    for row in range(BPW):
        pltpu.sync_copy(tok_idx_hbm.at[wid * BPW + row],
                        idx_all.at[pl.ds(row * L, L)])
    pltpu.sync_copy(ttab_hbm, trow)
    t0s = [trow[pl.ds(c * 16, 16)] * SCALE for c in range(D // 16)]

    bufs = (base_a, base_b)
    obufs = (out_a, out_b)
    sems = (gsem_a, gsem_b)

    def fire(k, buf, sem):
        for s in range(NSUB):
            pltpu.async_copy(
                ttable_hbm.at[idx_all.at[pl.ds(k * CH + s * SUB, SUB)]],
                buf.at[pl.ds(s * SUB, SUB)], sem)

    def drain(buf, sem):
        for s in range(NSUB):
            pltpu.make_async_copy(
                ttable_hbm.at[idx_all.at[pl.ds(s * SUB, SUB)]],
                buf.at[pl.ds(s * SUB, SUB)], sem).wait()

    def handle_event(obuf, r, gi):
        bi = gi // L
        li = gi - bi * L
        pltpu.sync_copy(catf_hbm.at[bi, li], evcatf.at[pl.ds(0, NCAT)])
        lanes = lax.iota(jnp.int32, 16)
        raw = evcatf[pl.ds(0, 16)]
        evcat_idx[pl.ds(0, 16)] = jnp.where(lanes < NCAT,
                                            raw + lanes * CATK, 0)
        pltpu.async_copy(cflat_hbm.at[evcat_idx], evcats, esem).wait()
        pltpu.sync_copy(t8_hbm.at[bi, li], evt8)
        for c in range(D // 16):
            sl = pl.ds(c * 16, 16)
            acc = evcats[0, sl]
            for f in range(1, NCAT):
                acc = acc + evcats[f, sl]
            obuf[r, sl] = acc * SCALE + evt8[pl.ds(c * 16, 16)]

    def process(k, buf, obuf):
        def grp(g, _):
            loff = k * CH + g * 16
            tv = idx_all[pl.ds(loff, 16)]
            for j in range(16):
                r = g * 16 + j
                for c in range(D // 16):
                    sl = pl.ds(c * 16, 16)
                    obuf[r, sl] = buf[r, sl] * SCALE + t0s[c]
            anyev = tv[0] == SPECIAL_EVENT
            for j in range(1, 16):
                anyev = jnp.logical_or(anyev, tv[j] == SPECIAL_EVENT)

            @pl.when(anyev)
            def _():
                for j in range(16):
                    r = g * 16 + j

                    @pl.when(tv[j] == SPECIAL_EVENT)
                    def _():
                        handle_event(obuf, r, wbase + loff + j)

            return 0

        lax.fori_loop(0, NG, grp, 0)
        brow = wid * BPW + k * (CH // L)
        pltpu.sync_copy(obuf.at[pl.ds(0, L)], out_hbm.at[brow])
        pltpu.sync_copy(obuf.at[pl.ds(L, L)], out_hbm.at[brow + 1])

    fire(0, base_a, gsem_a)

    def outer(k0, _):
        for b in range(2):
            k = k0 * 2 + b

            @pl.when(k + 1 < NCHUNK)
            def _():
                fire(k + 1, bufs[1 - b], sems[1 - b])

            drain(bufs[b], sems[b])
            process(k, bufs[b], obufs[b])
        return 0

    lax.fori_loop(0, NCHUNK // 2, outer, 0)


_sc_combine = functools.partial(
    pl.kernel,
    mesh=plsc.VectorSubcoreMesh(core_axis_name="c", subcore_axis_name="s"),
    compiler_params=pltpu.CompilerParams(use_tc_tiling_on_sc=False),
    out_type=jax.ShapeDtypeStruct((B, L, D), jnp.float32),
    scratch_types=[
        pltpu.VMEM((TOK_PER_W,), jnp.int32),
        pltpu.VMEM((CH, D), jnp.float32),
        pltpu.VMEM((CH, D), jnp.float32),
        pltpu.VMEM((CH, D), jnp.float32),
        pltpu.VMEM((CH, D), jnp.float32),
        pltpu.VMEM((16,), jnp.int32),
        pltpu.VMEM((16,), jnp.int32),
        pltpu.VMEM((16, D), jnp.float32),
        pltpu.VMEM((D,), jnp.float32),
        pltpu.VMEM((2 * D,), jnp.float32),
        pltpu.SemaphoreType.DMA,
        pltpu.SemaphoreType.DMA,
        pltpu.SemaphoreType.DMA,
    ],
)(_sc_body)


def kernel(token_ids, cat_feats, num_feats, time_feats, token_table, cat_tables,
           W1, b1, W2, b2, Wt1, bt1, Wt2, bt2, type_table):
    cflat = cat_tables.reshape(NCAT * CATK, D)
    t8 = _mlp_t8(num_feats, time_feats,
                 W1, b1.reshape(1, D), W2, b2.reshape(1, D),
                 Wt1, bt1.reshape(1, D), Wt2, bt2.reshape(1, D), type_table)
    return _sc_combine(token_ids.astype(jnp.int32),
                       cat_feats.astype(jnp.int32), t8,
                       token_table, cflat, type_table.reshape(-1))


# speedup vs baseline: 1.1282x; 1.1282x over previous
import functools
import math

import jax
import jax.numpy as jnp
from jax import lax
from jax.experimental import pallas as pl
from jax.experimental.pallas import tpu as pltpu
from jax.experimental.pallas import tpu_sc as plsc

SPECIAL_EVENT = 1
V, D, NCAT, CATK, NNUM, NTIME, B, L = 1000000, 64, 8, 1000, 16, 8, 1024, 200
BL = B * L
SCALE = math.sqrt(D)

NC, NS = 2, 16
NW = NC * NS
TOK_PER_W = BL // NW
CH = 400
NCHUNK = TOK_PER_W // CH
NG = CH // 16
SUB = 80
NSUB = CH // SUB
BPW = B // NW


_RB = 12800


def _mlp_body(nf, tf, w1, b1, w2, b2, wt1, bt1, wt2, bt2, ttab, out):
    h = jnp.maximum(
        jnp.dot(nf[...], w1[...], preferred_element_type=jnp.float32) + b1[...], 0.0)
    num = jnp.dot(h, w2[...], preferred_element_type=jnp.float32) + b2[...]
    ht = jnp.maximum(
        jnp.dot(tf[...], wt1[...], preferred_element_type=jnp.float32) + bt1[...], 0.0)
    tim = jnp.dot(ht, wt2[...], preferred_element_type=jnp.float32) + bt2[...]
    out[...] = (num + tim + ttab[1:2, :]) * SCALE


def _mlp_t8(nf, tf, W1, b1, W2, b2, Wt1, bt1, Wt2, bt2, ttab):
    grid = (BL // _RB,)
    full = lambda i: (0, 0)
    return pl.pallas_call(
        _mlp_body,
        grid=grid,
        in_specs=[
            pl.BlockSpec((_RB, NNUM), lambda i: (i, 0)),
            pl.BlockSpec((_RB, NTIME), lambda i: (i, 0)),
            pl.BlockSpec((NNUM, D), full),
            pl.BlockSpec((1, D), full),
            pl.BlockSpec((D, D), full),
            pl.BlockSpec((1, D), full),
            pl.BlockSpec((NTIME, D), full),
            pl.BlockSpec((1, D), full),
            pl.BlockSpec((D, D), full),
            pl.BlockSpec((1, D), full),
            pl.BlockSpec((2, D), full),
        ],
        out_specs=pl.BlockSpec((_RB, D), lambda i: (i, 0)),
        out_shape=jax.ShapeDtypeStruct((BL, D), jnp.float32),
    )(nf, tf, W1, b1, W2, b2, Wt1, bt1, Wt2, bt2, ttab)


def _sc_body(tok_idx_hbm, catf_hbm, t8_hbm, ttable_hbm, cflat_hbm, ttab_hbm,
             out_hbm, idx_all, base_a, base_b, evcatf, evcat_idx, evcats,
             evt8, trow, gsem_a, gsem_b, esem):
    wid = lax.axis_index("s") * NC + lax.axis_index("c")
    wbase = wid * TOK_PER_W

    pltpu.sync_copy(tok_idx_hbm.at[pl.ds(wbase, TOK_PER_W)], idx_all)
    pltpu.sync_copy(ttab_hbm, trow)
    t0s = [trow[pl.ds(c * 16, 16)] * SCALE for c in range(D // 16)]

    bufs = (base_a, base_b)
    sems = (gsem_a, gsem_b)

    def fire(k, buf, sem):
        for s in range(NSUB):
            pltpu.async_copy(
                ttable_hbm.at[idx_all.at[pl.ds(k * CH + s * SUB, SUB)]],
                buf.at[pl.ds(s * SUB, SUB)], sem)

    def drain(buf, sem):
        for s in range(NSUB):
            pltpu.make_async_copy(
                ttable_hbm.at[idx_all.at[pl.ds(s * SUB, SUB)]],
                buf.at[pl.ds(s * SUB, SUB)], sem).wait()

    def handle_event(buf, r, gi):
        bi = gi // L
        li = gi - bi * L
        pltpu.sync_copy(catf_hbm.at[bi, li], evcatf.at[pl.ds(0, NCAT)])
        lanes = lax.iota(jnp.int32, 16)
        raw = evcatf[pl.ds(0, 16)]
        evcat_idx[pl.ds(0, 16)] = jnp.where(lanes < NCAT,
                                            raw + lanes * CATK, 0)
        pltpu.async_copy(cflat_hbm.at[evcat_idx], evcats, esem).wait()
        pltpu.sync_copy(t8_hbm.at[pl.ds(gi, 1)], evt8)
        for c in range(D // 16):
            sl = pl.ds(c * 16, 16)
            acc = evcats[0, sl]
            for f in range(1, NCAT):
                acc = acc + evcats[f, sl]
            buf[r, sl] = acc * SCALE + evt8[0, pl.ds(c * 16, 16)]

    def process(k, buf):
        def grp(g, _):
            loff = k * CH + g * 16
            tv = idx_all[pl.ds(loff, 16)]
            for j in range(16):
                r = g * 16 + j
                for c in range(D // 16):
                    sl = pl.ds(c * 16, 16)
                    buf[r, sl] = buf[r, sl] * SCALE + t0s[c]
            anyev = tv[0] == SPECIAL_EVENT
            for j in range(1, 16):
                anyev = jnp.logical_or(anyev, tv[j] == SPECIAL_EVENT)

            @pl.when(anyev)
            def _():
                for j in range(16):
                    r = g * 16 + j

                    @pl.when(tv[j] == SPECIAL_EVENT)
                    def _():
                        handle_event(buf, r, wbase + loff + j)

            return 0

        lax.fori_loop(0, NG, grp, 0)
        brow = wid * BPW + k * (CH // L)
        pltpu.sync_copy(buf.at[pl.ds(0, L)], out_hbm.at[brow])
        pltpu.sync_copy(buf.at[pl.ds(L, L)], out_hbm.at[brow + 1])

    fire(0, base_a, gsem_a)

    def outer(k0, _):
        for b in range(2):
            k = k0 * 2 + b

            @pl.when(k + 1 < NCHUNK)
            def _():
                fire(k + 1, bufs[1 - b], sems[1 - b])

            drain(bufs[b], sems[b])
            process(k, bufs[b])
        return 0

    lax.fori_loop(0, NCHUNK // 2, outer, 0)


_sc_combine = functools.partial(
    pl.kernel,
    mesh=plsc.VectorSubcoreMesh(core_axis_name="c", subcore_axis_name="s"),
    compiler_params=pltpu.CompilerParams(use_tc_tiling_on_sc=False),
    out_type=jax.ShapeDtypeStruct((B, L, D), jnp.float32),
    scratch_types=[
        pltpu.VMEM((TOK_PER_W,), jnp.int32),
        pltpu.VMEM((CH, D), jnp.float32),
        pltpu.VMEM((CH, D), jnp.float32),
        pltpu.VMEM((16,), jnp.int32),
        pltpu.VMEM((16,), jnp.int32),
        pltpu.VMEM((16, D), jnp.float32),
        pltpu.VMEM((1, D), jnp.float32),
        pltpu.VMEM((2 * D,), jnp.float32),
        pltpu.SemaphoreType.DMA,
        pltpu.SemaphoreType.DMA,
        pltpu.SemaphoreType.DMA,
    ],
)(_sc_body)


def kernel(token_ids, cat_feats, num_feats, time_feats, token_table, cat_tables,
           W1, b1, W2, b2, Wt1, bt1, Wt2, bt2, type_table):
    tok_flat = token_ids.reshape(BL).astype(jnp.int32)
    cflat = cat_tables.reshape(NCAT * CATK, D)
    t8 = _mlp_t8(num_feats.reshape(BL, NNUM), time_feats.reshape(BL, NTIME),
                 W1, b1.reshape(1, D), W2, b2.reshape(1, D),
                 Wt1, bt1.reshape(1, D), Wt2, bt2.reshape(1, D), type_table)
    return _sc_combine(tok_flat, cat_feats.astype(jnp.int32), t8,
                       token_table, cflat, type_table.reshape(-1))
